# baseline (speedup 1.0000x reference)
import functools

import jax
import jax.numpy as jnp
from jax.experimental import pallas as pl
from jax.experimental.pallas import tpu as pltpu

DIM = 2304
CBD = 4
NCODES = 625
NQ = 256
TB = 512


def _rvq_kernel(x_ref, wi_ref, bi_ref, wo_ref, bo_ref, cbt_ref,
                qout_ref, idx_ref, resid_ref):
    resid_ref[...] = x_ref[...]
    qout_ref[...] = jnp.zeros_like(qout_ref)

    cbt = cbt_ref[...]

    HB = TB // 2
    def layer(l, _):
        wi = wi_ref[l]
        wo = wo_ref[l]
        bi = bi_ref[pl.ds(l, 1), :]
        bo = bo_ref[pl.ds(l, 1), :]
        wo_f = wo.astype(jnp.float32)

        def half(h):
            rows = slice(h * HB, (h + 1) * HB)
            resid = resid_ref[rows, :]
            lin = jax.lax.dot_general(
                resid.astype(jnp.bfloat16), wi, (((1,), (1,)), ((), ())),
                preferred_element_type=jnp.float32)
            q = lin + bi

            s0 = q[:, 0:1] * q[:, 0:1]
            s1 = q[:, 1:2] * q[:, 1:2]
            s2 = q[:, 2:3] * q[:, 2:3]
            s3 = q[:, 3:4] * q[:, 3:4]
            nsq = (s0 + s2) + (s1 + s3)
            norm = jnp.sqrt(nsq)
            qn = q / norm

            cos = jax.lax.dot_general(
                qn, cbt, (((1,), (0,)), ((), ())),
                preferred_element_type=jnp.float32)

            cmax = jnp.max(cos, axis=1, keepdims=True)
            lane = jax.lax.broadcasted_iota(jnp.int32, (HB, NCODES), 1)
            hit = cos == cmax
            idx = jnp.min(jnp.where(hit, lane, NCODES), axis=1)

            sel = (lane == idx[:, None]).astype(jnp.float32)
            cvec = jax.lax.dot_general(
                sel, cbt, (((1,), (1,)), ((), ())),
                preferred_element_type=jnp.float32)

            NC = 256
            for c in range(DIM // NC):
                sl = slice(c * NC, (c + 1) * NC)
                dec_c = jax.lax.dot_general(
                    cvec, wo_f[:, sl], (((1,), (0,)), ((), ())),
                    preferred_element_type=jnp.float32)
                dec_c = dec_c + bo[:, sl]
                resid_ref[rows, sl] = resid[:, sl] - dec_c
                qout_ref[rows, sl] = qout_ref[rows, sl] + dec_c
            idx_ref[pl.ds(l, 1), rows] = idx[None, :]

        half(0)
        half(1)
        return 0

    jax.lax.fori_loop(0, NQ, layer, 0, unroll=2)


def kernel(x, W_in, b_in, W_out, b_out, codebook):
    B, S, _ = x.shape
    T = B * S
    nb = T // TB

    x2 = x.reshape(T, DIM)
    wi_bf = jnp.swapaxes(W_in.astype(jnp.bfloat16), 1, 2)
    wo_bf = W_out.astype(jnp.bfloat16)
    cbt = codebook.T

    qout, idxs = pl.pallas_call(
        _rvq_kernel,
        grid=(nb,),
        in_specs=[
            pl.BlockSpec((TB, DIM), lambda i: (i, 0)),
            pl.BlockSpec((NQ, CBD, DIM), lambda i: (0, 0, 0)),
            pl.BlockSpec((NQ, CBD), lambda i: (0, 0)),
            pl.BlockSpec((NQ, CBD, DIM), lambda i: (0, 0, 0)),
            pl.BlockSpec((NQ, DIM), lambda i: (0, 0)),
            pl.BlockSpec((CBD, NCODES), lambda i: (0, 0)),
        ],
        out_specs=[
            pl.BlockSpec((TB, DIM), lambda i: (i, 0)),
            pl.BlockSpec((NQ, TB), lambda i: (0, i)),
        ],
        out_shape=[
            jax.ShapeDtypeStruct((T, DIM), jnp.float32),
            jax.ShapeDtypeStruct((NQ, T), jnp.int32),
        ],
        scratch_shapes=[pltpu.VMEM((TB, DIM), jnp.float32)],
    )(x2, wi_bf, b_in, wo_bf, b_out, cbt)

    quantized_out = qout.reshape(B, S, DIM)
    all_indices = jnp.transpose(idxs.reshape(NQ, B, S), (1, 2, 0))
    return quantized_out, all_indices


# speedup vs baseline: 1.4262x; 1.4262x over previous
import functools

import jax
import jax.numpy as jnp
from jax.experimental import pallas as pl
from jax.experimental.pallas import tpu as pltpu

DIM = 2304
CBD = 4
NCODES = 625
NQ = 256
TB = 1024


def _rvq_kernel(x_ref, wi_ref, bi_ref, wo_ref, bo_ref, cbt_ref,
                qout_hbm_ref, idx_ref, qout_ref, resid_ref, sem):
    i = pl.program_id(0)
    cp_in = pltpu.make_async_copy(
        x_ref.at[pl.ds(i * TB, TB), :], resid_ref, sem)
    cp_in.start()
    cp_in.wait()

    cbt = cbt_ref[...]

    def layer(l, qout_first):
        wi = wi_ref[l]
        wo = wo_ref[l]
        bi = bi_ref[pl.ds(l, 1), :]
        bo = bo_ref[pl.ds(l, 1), :]

        resid = resid_ref[...]
        lin = jax.lax.dot_general(
            resid.astype(jnp.bfloat16), wi, (((1,), (1,)), ((), ())),
            preferred_element_type=jnp.float32)
        q = lin + bi

        s0 = q[:, 0:1] * q[:, 0:1]
        s1 = q[:, 1:2] * q[:, 1:2]
        s2 = q[:, 2:3] * q[:, 2:3]
        s3 = q[:, 3:4] * q[:, 3:4]
        nsq = (s0 + s2) + (s1 + s3)
        norm = jnp.sqrt(nsq)
        qn = q / norm

        cos = jax.lax.dot_general(
            qn, cbt, (((1,), (0,)), ((), ())),
            preferred_element_type=jnp.float32)

        cmax = jnp.max(cos, axis=1, keepdims=True)
        lane = jax.lax.broadcasted_iota(jnp.int32, (TB, NCODES), 1)
        hit = cos == cmax
        idx = jnp.min(jnp.where(hit, lane, NCODES), axis=1)

        sel = (lane == idx[:, None]).astype(jnp.float32)
        cvec = jax.lax.dot_general(
            sel, cbt, (((1,), (1,)), ((), ())),
            preferred_element_type=jnp.float32)

        wo_f = wo.astype(jnp.float32)
        NC = 256
        for c in range(DIM // NC):
            sl = slice(c * NC, (c + 1) * NC)
            dec_c = jax.lax.dot_general(
                cvec, wo_f[:, sl], (((1,), (0,)), ((), ())),
                preferred_element_type=jnp.float32)
            dec_c = dec_c + bo[:, sl]
            resid_ref[:, sl] = resid[:, sl] - dec_c
            if qout_first:
                qout_ref[:, sl] = dec_c
            else:
                qout_ref[:, sl] = qout_ref[:, sl] + dec_c
        idx_ref[pl.ds(l, 1), :] = idx[None, :]

    layer(0, True)
    jax.lax.fori_loop(1, NQ, lambda l, c: (layer(l, False), 0)[1], 0, unroll=2)
    cp_out = pltpu.make_async_copy(
        qout_ref, qout_hbm_ref.at[pl.ds(i * TB, TB), :], sem)
    cp_out.start()
    cp_out.wait()


def kernel(x, W_in, b_in, W_out, b_out, codebook):
    B, S, _ = x.shape
    T = B * S
    nb = T // TB

    x2 = x.reshape(T, DIM)
    wi_bf = jnp.swapaxes(W_in.astype(jnp.bfloat16), 1, 2)
    wo_bf = W_out.astype(jnp.bfloat16)
    cbt = codebook.T

    qout, idxs = pl.pallas_call(
        _rvq_kernel,
        grid=(nb,),
        in_specs=[
            pl.BlockSpec(memory_space=pl.ANY),
            pl.BlockSpec((NQ, CBD, DIM), lambda i: (0, 0, 0)),
            pl.BlockSpec((NQ, CBD), lambda i: (0, 0)),
            pl.BlockSpec((NQ, CBD, DIM), lambda i: (0, 0, 0)),
            pl.BlockSpec((NQ, DIM), lambda i: (0, 0)),
            pl.BlockSpec((CBD, NCODES), lambda i: (0, 0)),
        ],
        out_specs=[
            pl.BlockSpec(memory_space=pl.ANY),
            pl.BlockSpec((NQ, TB), lambda i: (0, i)),
        ],
        out_shape=[
            jax.ShapeDtypeStruct((T, DIM), jnp.float32),
            jax.ShapeDtypeStruct((NQ, T), jnp.int32),
        ],
        scratch_shapes=[pltpu.VMEM((TB, DIM), jnp.float32),
                        pltpu.VMEM((TB, DIM), jnp.float32),
                        pltpu.SemaphoreType.DMA],
    )(x2, wi_bf, b_in, wo_bf, b_out, cbt)

    quantized_out = qout.reshape(B, S, DIM)
    all_indices = jnp.transpose(idxs.reshape(NQ, B, S), (1, 2, 0))
    return quantized_out, all_indices
